# baseline (speedup 1.0000x reference)
import functools
import math

import jax
import jax.numpy as jnp
import numpy as np
from jax import lax
from jax.experimental import pallas as pl
from jax.experimental.pallas import tpu as pltpu
from jax.experimental.pallas import tpu_sc as plsc

_LANES = 16
_NUM_WORKERS = 32


def _pair_perm(d):
    perm = np.empty(d, dtype=np.int32)
    for j in range(d // 32):
        for i in range(16):
            perm[32 * j + 2 * i] = 32 * j + i
            perm[32 * j + 2 * i + 1] = 32 * j + 16 + i
    return perm


def kernel(input_ids, token_table, pos_table):
    B, S = input_ids.shape
    V, D = token_table.shape
    N = B * S
    scale = math.sqrt(float(D))
    s_per_w = S // _NUM_WORKERS
    npair = D // (2 * _LANES)

    pos_shuf = jnp.take(pos_table, jnp.asarray(_pair_perm(D)), axis=1)
    pos_pk = lax.bitcast_convert_type(
        pos_shuf.astype(jnp.bfloat16).reshape(S, D // 2, 2), jnp.int32)

    mesh = plsc.VectorSubcoreMesh(core_axis_name="c", subcore_axis_name="s")

    @functools.partial(
        pl.kernel,
        mesh=mesh,
        out_type=jax.ShapeDtypeStruct((N, D), jnp.float32),
        scratch_types=[
            pltpu.VMEM((B, s_per_w), jnp.int32),
            pltpu.VMEM((s_per_w, D // 2), jnp.int32),
            pltpu.VMEM((s_per_w, D), jnp.float32),
            pltpu.VMEM((s_per_w, D), jnp.float32),
            pltpu.SemaphoreType.DMA,
            pltpu.SemaphoreType.DMA,
            pltpu.SemaphoreType.DMA,
            pltpu.SemaphoreType.DMA,
            pltpu.SemaphoreType.DMA,
            pltpu.SemaphoreType.DMA,
        ],
    )
    def body(ids_hbm, tok_hbm, pos_hbm, out_hbm, idx_v, pos_v, t0, t1,
             g0, g1, o0, o1, isem, psem):
        wid = lax.axis_index("s") * 2 + lax.axis_index("c")
        s0 = wid * s_per_w
        idx_cps = [
            pltpu.async_copy(ids_hbm.at[pl.ds(b * S + s0, s_per_w)],
                             idx_v.at[b], isem)
            for b in range(B)
        ]
        pos_cp = pltpu.async_copy(pos_hbm.at[pl.ds(s0, s_per_w)], pos_v, psem)
        for cp in idx_cps:
            cp.wait()

        tbufs = [t0, t1]
        gsems = [g0, g1]
        osems = [o0, o1]
        gathers = [None, None]
        stores = [None, None]
        gathers[0] = pltpu.async_copy(tok_hbm.at[idx_v.at[0]], t0, g0)
        pos_cp.wait()
        for b in range(B):
            cur = b % 2
            nxt = (b + 1) % 2
            if b + 1 < B:
                if stores[nxt] is not None:
                    stores[nxt].wait()
                gathers[nxt] = pltpu.async_copy(
                    tok_hbm.at[idx_v.at[b + 1]], tbufs[nxt], gsems[nxt])
            gathers[cur].wait()
            buf = tbufs[cur]

            def row(r, _, buf=buf):
                for k in range(npair):
                    pw = pos_v[r, pl.ds(k * _LANES, _LANES)]
                    p_even = lax.bitcast_convert_type(pw << 16, jnp.float32)
                    p_odd = lax.bitcast_convert_type(pw & (-65536),
                                                     jnp.float32)
                    sl_a = pl.ds(k * 2 * _LANES, _LANES)
                    sl_b = pl.ds(k * 2 * _LANES + _LANES, _LANES)
                    buf[r, sl_a] = buf[r, sl_a] * scale + p_even
                    buf[r, sl_b] = buf[r, sl_b] * scale + p_odd
                return 0

            lax.fori_loop(0, s_per_w, row, 0)
            stores[cur] = pltpu.async_copy(
                buf, out_hbm.at[pl.ds(b * S + s0, s_per_w)], osems[cur])
        stores[0].wait()
        stores[1].wait()

    out = body(input_ids.reshape(N), token_table, pos_pk)
    return out.reshape(B, S, D)


# speedup vs baseline: 1.3871x; 1.3871x over previous
import functools
import math

import jax
import jax.numpy as jnp
import numpy as np
from jax import lax
from jax.experimental import pallas as pl
from jax.experimental.pallas import tpu as pltpu
from jax.experimental.pallas import tpu_sc as plsc

_LANES = 16
_NUM_WORKERS = 32


def kernel(input_ids, token_table, pos_table):
    B, S = input_ids.shape
    V, D = token_table.shape
    N = B * S
    scale = math.sqrt(float(D))
    s_per_w = S // _NUM_WORKERS
    npair = D // (2 * _LANES)

    pos_bf = pos_table.astype(jnp.bfloat16)
    pos_pk = lax.bitcast_convert_type(
        jnp.stack([pos_bf[:, :D // 2], pos_bf[:, D // 2:]], axis=-1),
        jnp.int32)

    mesh = plsc.VectorSubcoreMesh(core_axis_name="c", subcore_axis_name="s")

    @functools.partial(
        pl.kernel,
        mesh=mesh,
        out_type=jax.ShapeDtypeStruct((N, D), jnp.float32),
        scratch_types=[
            pltpu.VMEM((B, s_per_w), jnp.int32),
            pltpu.VMEM((s_per_w, D // 2), jnp.int32),
            pltpu.VMEM((s_per_w, D), jnp.float32),
            pltpu.VMEM((s_per_w, D), jnp.float32),
            pltpu.SemaphoreType.DMA,
            pltpu.SemaphoreType.DMA,
            pltpu.SemaphoreType.DMA,
            pltpu.SemaphoreType.DMA,
            pltpu.SemaphoreType.DMA,
            pltpu.SemaphoreType.DMA,
        ],
    )
    def body(ids_hbm, tok_hbm, pos_hbm, out_hbm, idx_v, pos_v, t0, t1,
             g0, g1, o0, o1, isem, psem):
        wid = lax.axis_index("s") * 2 + lax.axis_index("c")
        s0 = wid * s_per_w
        idx_cps = [
            pltpu.async_copy(ids_hbm.at[pl.ds(b * S + s0, s_per_w)],
                             idx_v.at[b], isem)
            for b in range(B)
        ]
        pos_cp = pltpu.async_copy(pos_hbm.at[pl.ds(s0, s_per_w)], pos_v, psem)
        for cp in idx_cps:
            cp.wait()

        tbufs = [t0, t1]
        gsems = [g0, g1]
        osems = [o0, o1]
        gathers = [None, None]
        stores = [None, None]
        gathers[0] = pltpu.async_copy(tok_hbm.at[idx_v.at[0]], t0, g0)
        pos_cp.wait()
        for b in range(B):
            cur = b % 2
            nxt = (b + 1) % 2
            if b + 1 < B:
                if stores[nxt] is not None:
                    stores[nxt].wait()
                gathers[nxt] = pltpu.async_copy(
                    tok_hbm.at[idx_v.at[b + 1]], tbufs[nxt], gsems[nxt])
            gathers[cur].wait()
            buf = tbufs[cur]

            def row(r, _, buf=buf):
                for k in range(npair):
                    pw = pos_v[r, pl.ds(k * _LANES, _LANES)]
                    p_even = lax.bitcast_convert_type(pw << 16, jnp.float32)
                    p_odd = lax.bitcast_convert_type(pw & (-65536),
                                                     jnp.float32)
                    sl_a = pl.ds(k * _LANES, _LANES)
                    sl_b = pl.ds(D // 2 + k * _LANES, _LANES)
                    buf[r, sl_a] = buf[r, sl_a] * scale + p_even
                    buf[r, sl_b] = buf[r, sl_b] * scale + p_odd
                return 0

            lax.fori_loop(0, s_per_w, row, 0)
            stores[cur] = pltpu.async_copy(
                buf, out_hbm.at[pl.ds(b * S + s0, s_per_w)], osems[cur])
        stores[0].wait()
        stores[1].wait()

    out = body(input_ids.reshape(N), token_table, pos_pk)
    return out.reshape(B, S, D)


# speedup vs baseline: 1.9728x; 1.4222x over previous
import functools
import math

import jax
import jax.numpy as jnp
from jax import lax
from jax.experimental import pallas as pl
from jax.experimental.pallas import tpu as pltpu
from jax.experimental.pallas import tpu_sc as plsc

_LANES = 16
_NUM_WORKERS = 32


def kernel(input_ids, token_table, pos_table):
    B, S = input_ids.shape
    V, D = token_table.shape
    N = B * S
    scale = math.sqrt(float(D))
    s_per_w = S // _NUM_WORKERS
    nvec = D // _LANES

    mesh = plsc.VectorSubcoreMesh(core_axis_name="c", subcore_axis_name="s")

    @functools.partial(
        pl.kernel,
        mesh=mesh,
        out_type=jax.ShapeDtypeStruct((N, D), jnp.float32),
        scratch_types=[
            pltpu.VMEM((B, s_per_w), jnp.int32),
            pltpu.VMEM((s_per_w, D), jnp.float32),
            pltpu.VMEM((s_per_w, D), jnp.float32),
            pltpu.VMEM((s_per_w, D), jnp.float32),
            pltpu.SemaphoreType.DMA,
            pltpu.SemaphoreType.DMA,
            pltpu.SemaphoreType.DMA,
            pltpu.SemaphoreType.DMA,
            pltpu.SemaphoreType.DMA,
            pltpu.SemaphoreType.DMA,
        ],
    )
    def body(ids_hbm, tok_hbm, pos_hbm, out_hbm, idx_v, pos_v, t0, t1,
             g0, g1, o0, o1, isem, psem):
        wid = lax.axis_index("s") * 2 + lax.axis_index("c")
        s0 = wid * s_per_w
        idx_cps = [
            pltpu.async_copy(ids_hbm.at[pl.ds(b * S + s0, s_per_w)],
                             idx_v.at[b], isem)
            for b in range(B)
        ]
        pos_cp = pltpu.async_copy(pos_hbm.at[pl.ds(s0, s_per_w)], pos_v, psem)
        for cp in idx_cps:
            cp.wait()

        tbufs = [t0, t1]
        gsems = [g0, g1]
        osems = [o0, o1]
        gathers = [None, None]
        stores = [None, None]
        gathers[0] = pltpu.async_copy(tok_hbm.at[idx_v.at[0]], t0, g0)
        pos_cp.wait()
        for b in range(B):
            cur = b % 2
            nxt = (b + 1) % 2
            if b + 1 < B:
                if stores[nxt] is not None:
                    stores[nxt].wait()
                gathers[nxt] = pltpu.async_copy(
                    tok_hbm.at[idx_v.at[b + 1]], tbufs[nxt], gsems[nxt])
            gathers[cur].wait()
            buf = tbufs[cur]

            def row(r, _, buf=buf):
                for k in range(nvec):
                    sl = pl.ds(k * _LANES, _LANES)
                    buf[r, sl] = buf[r, sl] * scale + pos_v[r, sl]
                return 0

            lax.fori_loop(0, s_per_w, row, 0)
            stores[cur] = pltpu.async_copy(
                buf, out_hbm.at[pl.ds(b * S + s0, s_per_w)], osems[cur])
        stores[0].wait()
        stores[1].wait()

    out = body(input_ids.reshape(N), token_table, pos_table)
    return out.reshape(B, S, D)
